# baseline (speedup 1.0000x reference)
import functools

import jax
import jax.numpy as jnp
from jax import lax
from jax.experimental import pallas as pl
from jax.experimental.pallas import tpu as pltpu
from jax.experimental.pallas import tpu_sc as plsc

B = 16384
DMF = 16
DMLP = 32
NC = 2
NS = 16
NW = NC * NS
BPW = B // NW
CHUNK = 128
NCHUNK = BPW // CHUNK


def _sc_gather_body(uidx_hbm, iidx_hbm, mfu_hbm, mfi_hbm, mlpu_hbm, mlpi_hbm,
                    out_mlpu, out_mlpi, out_mfp,
                    uidx_v, iidx_v, mlpu_v, mlpi_v, mfu_v, mfi_v, sem):
    wid = lax.axis_index("s") * NC + lax.axis_index("c")
    base = wid * BPW
    crow = wid * NCHUNK
    pltpu.sync_copy(uidx_hbm.at[pl.ds(crow, NCHUNK)], uidx_v)
    pltpu.sync_copy(iidx_hbm.at[pl.ds(crow, NCHUNK)], iidx_v)
    copies = []
    for j in range(NCHUNK):
        dst = pl.ds(j * CHUNK, CHUNK)
        copies.append(pltpu.async_copy(mlpu_hbm.at[uidx_v.at[j]], mlpu_v.at[dst], sem))
        copies.append(pltpu.async_copy(mlpi_hbm.at[iidx_v.at[j]], mlpi_v.at[dst], sem))
        copies.append(pltpu.async_copy(mfu_hbm.at[uidx_v.at[j]], mfu_v.at[dst], sem))
        copies.append(pltpu.async_copy(mfi_hbm.at[iidx_v.at[j]], mfi_v.at[dst], sem))
    for c in copies:
        c.wait()
    def body(i, _):
        mfu_v[i, :] = mfu_v[i, :] * mfi_v[i, :]
        return 0
    lax.fori_loop(0, BPW, body, 0)
    pltpu.sync_copy(mlpu_v, out_mlpu.at[pl.ds(base, BPW)])
    pltpu.sync_copy(mlpi_v, out_mlpi.at[pl.ds(base, BPW)])
    pltpu.sync_copy(mfu_v, out_mfp.at[pl.ds(base, BPW)])


_sc_gather = functools.partial(
    pl.kernel,
    mesh=plsc.VectorSubcoreMesh(core_axis_name="c", subcore_axis_name="s"),
    out_type=[
        jax.ShapeDtypeStruct((B, DMLP), jnp.float32),
        jax.ShapeDtypeStruct((B, DMLP), jnp.float32),
        jax.ShapeDtypeStruct((B, DMF), jnp.float32),
    ],
    scratch_types=[
        pltpu.VMEM((NCHUNK, CHUNK), jnp.int32),
        pltpu.VMEM((NCHUNK, CHUNK), jnp.int32),
        pltpu.VMEM((BPW, DMLP), jnp.float32),
        pltpu.VMEM((BPW, DMLP), jnp.float32),
        pltpu.VMEM((BPW, DMF), jnp.float32),
        pltpu.VMEM((BPW, DMF), jnp.float32),
        pltpu.SemaphoreType.DMA,
    ],
    compiler_params=pltpu.CompilerParams(use_tc_tiling_on_sc=False),
)(_sc_gather_body)


def _mlp_body(mlpu_ref, mlpi_ref, mfp_ref, w0u_ref, w0i_ref, b0_ref,
              w1_ref, b1_ref, w2_ref, b2_ref, wnm_ref, wnh_ref, bn_ref,
              out_ref):
    xu = mlpu_ref[...]
    xi = mlpi_ref[...]
    h = jnp.dot(xu, w0u_ref[...], preferred_element_type=jnp.float32)
    h += jnp.dot(xi, w0i_ref[...], preferred_element_type=jnp.float32)
    h = jnp.maximum(h + b0_ref[...], 0.0)
    h = jnp.maximum(jnp.dot(h, w1_ref[...], preferred_element_type=jnp.float32)
                    + b1_ref[...], 0.0)
    h = jnp.maximum(jnp.dot(h, w2_ref[...], preferred_element_type=jnp.float32)
                    + b2_ref[...], 0.0)
    logit = jnp.dot(mfp_ref[...], wnm_ref[...], preferred_element_type=jnp.float32)
    logit += jnp.dot(h, wnh_ref[...], preferred_element_type=jnp.float32)
    logit += bn_ref[...]
    out_ref[...] = 1.0 / (1.0 + jnp.exp(-logit))


def _mlp_call(mlpu, mlpi, mfp, w0u, w0i, b0, w1, b1, w2, b2, wnm, wnh, bn):
    BT = 2048
    grid = (B // BT,)
    row_spec = lambda d: pl.BlockSpec((BT, d), lambda i: (i, 0))
    full = pl.BlockSpec(lambda i: (0, 0))
    return pl.pallas_call(
        _mlp_body,
        grid=grid,
        in_specs=[
            row_spec(DMLP), row_spec(DMLP), row_spec(DMF),
            pl.BlockSpec((DMLP, 32), lambda i: (0, 0)),
            pl.BlockSpec((DMLP, 32), lambda i: (0, 0)),
            pl.BlockSpec((1, 32), lambda i: (0, 0)),
            pl.BlockSpec((32, 16), lambda i: (0, 0)),
            pl.BlockSpec((1, 16), lambda i: (0, 0)),
            pl.BlockSpec((16, 8), lambda i: (0, 0)),
            pl.BlockSpec((1, 8), lambda i: (0, 0)),
            pl.BlockSpec((DMF, 1), lambda i: (0, 0)),
            pl.BlockSpec((8, 1), lambda i: (0, 0)),
            pl.BlockSpec((1, 1), lambda i: (0, 0)),
        ],
        out_specs=pl.BlockSpec((BT, 1), lambda i: (i, 0)),
        out_shape=jax.ShapeDtypeStruct((B, 1), jnp.float32),
    )(mlpu, mlpi, mfp, w0u, w0i, b0, w1, b1, w2, b2, wnm, wnh, bn)


@jax.jit
def kernel(user_indices, item_indices, mf_user_table, mf_item_table,
           mlp_user_table, mlp_item_table, W0, b0, W1, b1, W2, b2, Wn, bn):
    uidx = user_indices.astype(jnp.int32).reshape(B // CHUNK, CHUNK)
    iidx = item_indices.astype(jnp.int32).reshape(B // CHUNK, CHUNK)
    mlpu, mlpi, mfp = _sc_gather(uidx, iidx, mf_user_table, mf_item_table,
                                 mlp_user_table, mlp_item_table)
    return _mlp_call(mlpu, mlpi, mfp,
                     W0[:DMLP], W0[DMLP:], b0.reshape(1, 32),
                     W1, b1.reshape(1, 16), W2, b2.reshape(1, 8),
                     Wn[:DMF], Wn[DMF:], bn.reshape(1, 1))
